# baseline (speedup 1.0000x reference)
import functools

import jax
import jax.numpy as jnp
import numpy as np
from jax import lax
from jax.experimental import pallas as pl
from jax.experimental.pallas import tpu as pltpu
from jax.experimental.pallas import tpu_sc as plsc

N = 10000
E = 320000
D = 128
H = 8
LW = 16
NC, NS = 2, 16
NW = NC * NS
EPW = E // NW
RPS = 624
TAIL = N - NS * RPS

TB = 1000
TG = N // TB

_F32 = jnp.float32


def _sds(shape):
    return jax.ShapeDtypeStruct(shape, _F32)


def _dense1_body(x_ref, w_ref, avs_ref, avd_ref, h_ref, as_ref, ad_ref):
    h = jnp.dot(x_ref[...], w_ref[...], preferred_element_type=_F32)
    h_ref[...] = h
    as_ref[...] = jnp.dot(h, avs_ref[...], preferred_element_type=_F32)
    ad_ref[...] = jnp.dot(h, avd_ref[...], preferred_element_type=_F32)


def _dense1(x, W1, avs, avd):
    return pl.pallas_call(
        _dense1_body,
        grid=(TG,),
        in_specs=[
            pl.BlockSpec((TB, D), lambda i: (i, 0)),
            pl.BlockSpec((D, D), lambda i: (0, 0)),
            pl.BlockSpec((D, D), lambda i: (0, 0)),
            pl.BlockSpec((D, D), lambda i: (0, 0)),
        ],
        out_specs=[
            pl.BlockSpec((TB, D), lambda i: (i, 0)),
            pl.BlockSpec((TB, D), lambda i: (i, 0)),
            pl.BlockSpec((TB, D), lambda i: (i, 0)),
        ],
        out_shape=[_sds((N, D)), _sds((N, D)), _sds((N, D))],
    )(x, W1, avs, avd)


def _dense2_body(accp_ref, sp_ref, ones_ref, b1_ref, w2_ref, as2_ref,
                 ad2_ref, g2_ref, h2_ref, as_ref, ad_ref):
    acc = accp_ref[0] + accp_ref[1]
    s = jnp.dot(sp_ref[...], ones_ref[...], preferred_element_type=_F32)
    g = jnp.maximum(acc / (s + 1e-16) + b1_ref[...], 0.0)
    h2 = jnp.dot(g, w2_ref[...], preferred_element_type=_F32)
    h2_ref[...] = h2
    avals = []
    dvals = []
    for h in range(H):
        h2h = h2[:, h * D:(h + 1) * D]
        avals.append(jnp.dot(h2h, as2_ref[:, h:h + 1],
                             preferred_element_type=_F32))
        dvals.append(jnp.dot(h2h, ad2_ref[:, h:h + 1],
                             preferred_element_type=_F32))
    a8 = jnp.concatenate(avals, axis=1)
    d8 = jnp.concatenate(dvals, axis=1)
    as_ref[...] = jnp.dot(a8, g2_ref[...], preferred_element_type=_F32)
    ad_ref[...] = jnp.dot(d8, g2_ref[...], preferred_element_type=_F32)


def _dense2(accp, sp, ones32, b1, W2, AS2, AD2, G2):
    return pl.pallas_call(
        _dense2_body,
        grid=(TG,),
        in_specs=[
            pl.BlockSpec((NC, TB, D), lambda i: (0, i, 0)),
            pl.BlockSpec((TB, NW), lambda i: (i, 0)),
            pl.BlockSpec((NW, 1), lambda i: (0, 0)),
            pl.BlockSpec((1, D), lambda i: (0, 0)),
            pl.BlockSpec((D, H * D), lambda i: (0, 0)),
            pl.BlockSpec((D, H), lambda i: (0, 0)),
            pl.BlockSpec((D, H), lambda i: (0, 0)),
            pl.BlockSpec((H, D), lambda i: (0, 0)),
        ],
        out_specs=[
            pl.BlockSpec((TB, H * D), lambda i: (i, 0)),
            pl.BlockSpec((TB, D), lambda i: (i, 0)),
            pl.BlockSpec((TB, D), lambda i: (i, 0)),
        ],
        out_shape=[_sds((N, H * D)), _sds((N, D)), _sds((N, D))],
    )(accp, sp, ones32, b1, W2, AS2, AD2, G2)


def _recip_body(sp_ref, sel_ref, g2_ref, r_ref):
    s8 = jnp.dot(sp_ref[...], sel_ref[...], preferred_element_type=_F32)
    r8 = 1.0 / (s8 + 1e-16)
    r_ref[...] = jnp.dot(r8, g2_ref[...], preferred_element_type=_F32)


def _recip(spT, SEL, G2):
    return pl.pallas_call(
        _recip_body,
        grid=(TG,),
        in_specs=[
            pl.BlockSpec((TB, NW * H), lambda i: (i, 0)),
            pl.BlockSpec((NW * H, H), lambda i: (0, 0)),
            pl.BlockSpec((H, D), lambda i: (0, 0)),
        ],
        out_specs=pl.BlockSpec((TB, D), lambda i: (i, 0)),
        out_shape=_sds((N, D)),
    )(spT, SEL, G2)


def _final_body(accp_ref, b2_ref, o_ref):
    o_ref[...] = (accp_ref[0] + accp_ref[1]) * (1.0 / H) + b2_ref[...]


def _final(accp, b2):
    return pl.pallas_call(
        _final_body,
        grid=(TG,),
        in_specs=[
            pl.BlockSpec((NC, TB, D), lambda i: (0, i, 0)),
            pl.BlockSpec((1, D), lambda i: (0, 0)),
        ],
        out_specs=pl.BlockSpec((TB, D), lambda i: (i, 0)),
        out_shape=_sds((N, D)),
    )(accp, b2)


_MESH = plsc.VectorSubcoreMesh(core_axis_name="c", subcore_axis_name="s")
_SC_PARAMS = pltpu.CompilerParams(needs_layout_passes=False)

_GDN = lax.GatherDimensionNumbers(offset_dims=(), collapsed_slice_dims=(0,),
                                  start_index_map=(0,))


def _worker():
    cid = lax.axis_index("c")
    sid = lax.axis_index("s")
    return cid, sid, sid * NC + cid


def _bcast_lane(v, l):
    idx = jnp.full((LW, 1), l, jnp.int32)
    return lax.gather(v, idx, _GDN, (1,),
                      mode=lax.GatherScatterMode.PROMISE_IN_BOUNDS)


def _edge_coeffs(asg, adg, exb, ch):
    def body(e, _):
        es = asg[e, pl.ds(0, LW)] + adg[e, pl.ds(0, LW)]
        es = jnp.maximum(es, es * 0.2)
        exb[pl.ds(pl.multiple_of(e * LW, LW), LW)] = jnp.exp(es)
        return 0

    lax.fori_loop(0, ch, body, 0)


def _dump_shared(cid, sid, shared, out_h):
    pltpu.sync_copy(shared.at[pl.ds(sid * RPS, RPS)],
                    out_h.at[cid, pl.ds(sid * RPS, RPS)])

    @pl.when(sid == 0)
    def _():
        pltpu.sync_copy(shared.at[pl.ds(NS * RPS, TAIL)],
                        out_h.at[cid, pl.ds(NS * RPS, TAIL)])


CH = 16
BLK = 512
NBLK = E // BLK


def _block_range(wid, total_blocks):
    q, r = divmod(total_blocks, NW)
    nblk = jnp.where(wid < r, q + 1, q)
    start = jnp.where(wid < r, (q + 1) * wid, q * wid + r)
    return start, nblk


def _pipeline(wid, blk, nchi, srcb, dstb, load_extra, start_gathers,
              wait_gathers, compute, start_store, wait_store, src_h, dst_h):
    start, nblk = _block_range(wid, E // blk)

    def step(c, b, base, prefetch):
        b2 = 1 - b
        if prefetch:
            if isinstance(c, int):
                if c >= 1:
                    wait_store(b2)
            else:
                @pl.when(c >= 1)
                def _():
                    wait_store(b2)
            start_gathers(c + 1, b2)

        wait_gathers(b)
        compute(c, b, base)
        start_store(c, b, base)

    def block(s, _):
        base = pl.multiple_of((start + s) * blk, 8)
        pltpu.sync_copy(src_h.at[pl.ds(base, blk)], srcb)
        pltpu.sync_copy(dst_h.at[pl.ds(base, blk)], dstb)
        load_extra(base)
        start_gathers(0, 0)

        def pair(cc, _):
            for k in range(2):
                step(cc * 2 + k, k, base, True)
            return 0

        lax.fori_loop(0, nchi // 2 - 1, pair, 0)
        step(nchi - 2, 0, base, True)
        step(nchi - 1, 1, base, False)
        wait_store(0)
        wait_store(1)
        return 0

    lax.fori_loop(0, nblk, block, 0)


def _l1_body(src_h, dst_h, h1_h, ast_h, adt_h, z128_h, zN_h, accp_h, sp_h,
             srcb, dstb, dstv, asg, adg, rows, gsem, ssem, s1l, acc_sh):
    cid, sid, wid = _worker()

    pltpu.sync_copy(zN_h, s1l)

    @pl.when(sid == 0)
    def _():
        pltpu.sync_copy(z128_h, acc_sh)

    plsc.subcore_barrier()

    lane0 = lax.iota(jnp.int32, LW) == 0

    def sslice(c):
        return srcb.at[pl.ds(pl.multiple_of(c * CH, 8), CH)]

    def dslice(c):
        return dstb.at[pl.ds(pl.multiple_of(c * CH, 8), CH)]

    def start_gathers(c, b):
        pltpu.async_copy(ast_h.at[sslice(c)], asg[b], gsem[b])
        pltpu.async_copy(adt_h.at[dslice(c)], adg[b], gsem[b])
        pltpu.async_copy(h1_h.at[sslice(c)], rows[b], gsem[b])

    def wait_gathers(b):
        pltpu.make_async_copy(ast_h.at[sslice(0)], asg[b], gsem[b]).wait()
        pltpu.make_async_copy(adt_h.at[dslice(0)], adg[b], gsem[b]).wait()
        pltpu.make_async_copy(h1_h.at[sslice(0)], rows[b], gsem[b]).wait()

    def compute(c, b, base):
        dst16 = dstb[pl.ds(pl.multiple_of(c * CH, 8), CH)]
        dstv[b][...] = dst16
        for l in range(CH):
            es = asg[b][l, pl.ds(0, LW)] + adg[b][l, pl.ds(0, LW)]
            es = jnp.maximum(es, es * 0.2)
            exv = jnp.exp(es)
            for j2 in range(D // LW):
                rows[b][l, pl.ds(j2 * LW, LW)] = (
                    rows[b][l, pl.ds(j2 * LW, LW)] * exv)
            db = _bcast_lane(dst16, l)
            plsc.addupdate_scatter(s1l, [db], exv, mask=lane0)

    def start_store(c, b, base):
        pltpu.async_copy(rows[b], acc_sh.at[dstv[b]], ssem[b], add=True)

    def wait_store(b):
        pltpu.make_async_copy(rows[b], acc_sh.at[dstv[b]], ssem[b]).wait()

    _pipeline(wid, BLK, BLK // CH, srcb, dstb, lambda base: None,
              start_gathers, wait_gathers, compute, start_store, wait_store,
              src_h, dst_h)
    plsc.subcore_barrier()

    _dump_shared(cid, sid, acc_sh, accp_h)
    pltpu.sync_copy(s1l, sp_h.at[wid])


@functools.partial(
    pl.kernel,
    out_type=(_sds((NC, N, D)), _sds((NW, N))),
    mesh=_MESH,
    scratch_types=[
        pltpu.VMEM((BLK,), jnp.int32),
        pltpu.VMEM((BLK,), jnp.int32),
        [pltpu.VMEM((CH,), jnp.int32) for _ in range(2)],
        [pltpu.VMEM((CH, D), _F32) for _ in range(2)],
        [pltpu.VMEM((CH, D), _F32) for _ in range(2)],
        [pltpu.VMEM((CH, D), _F32) for _ in range(2)],
        [pltpu.SemaphoreType.DMA for _ in range(2)],
        [pltpu.SemaphoreType.DMA for _ in range(2)],
        pltpu.VMEM((N,), _F32),
        pltpu.VMEM_SHARED((N, D), _F32),
    ],
    compiler_params=_SC_PARAMS,
)
def _l1_edge(src_h, dst_h, h1_h, ast_h, adt_h, z128_h, zN_h, accp_h, sp_h,
             *rest):
    _l1_body(src_h, dst_h, h1_h, ast_h, adt_h, z128_h, zN_h, accp_h, sp_h,
             *rest)


CHA = 64


def _l2a_body(src_h, dst_h, ast_h, adt_h, z8N_h, exh_h, sp_h,
              srcb, dstb, asg, adg, exb, gsem, esem, s2l):
    cid, sid, wid = _worker()

    pltpu.sync_copy(z8N_h, s2l)
    plsc.subcore_barrier()

    hoff = (lax.iota(jnp.int32, LW) % H) * N
    hmask = lax.iota(jnp.int32, LW) < H

    def sslice(c):
        return srcb.at[pl.ds(pl.multiple_of(c * CHA, 8), CHA)]

    def dslice(c):
        return dstb.at[pl.ds(pl.multiple_of(c * CHA, 8), CHA)]

    def start_gathers(c, b):
        pltpu.async_copy(ast_h.at[sslice(c)], asg[b], gsem[b])
        pltpu.async_copy(adt_h.at[dslice(c)], adg[b], gsem[b])

    def wait_gathers(b):
        pltpu.make_async_copy(ast_h.at[sslice(0)], asg[b], gsem[b]).wait()
        pltpu.make_async_copy(adt_h.at[dslice(0)], adg[b], gsem[b]).wait()

    def compute(c, b, base):
        for g in range(CHA // LW):
            dst16 = dstb[pl.ds(pl.multiple_of(c * CHA + g * LW, 8), LW)]
            for l in range(LW):
                e = g * LW + l
                es = asg[b][e, pl.ds(0, LW)] + adg[b][e, pl.ds(0, LW)]
                es = jnp.maximum(es, es * 0.2)
                exv = jnp.exp(es)
                exb[b][pl.ds(e * LW, LW)] = exv
                db = _bcast_lane(dst16, l)
                plsc.addupdate_scatter(s2l, [hoff + db], exv, mask=hmask)

    def exh_at(c, base):
        off = pl.multiple_of((base + c * CHA) * LW, 8)
        return exh_h.at[pl.ds(off, CHA * LW)]

    def start_store(c, b, base):
        pltpu.async_copy(exb[b], exh_at(c, base), esem[b])

    def wait_store(b):
        pltpu.make_async_copy(exb[b], exh_h.at[pl.ds(0, CHA * LW)],
                              esem[b]).wait()

    _pipeline(wid, BLK, BLK // CHA, srcb, dstb, lambda base: None,
              start_gathers, wait_gathers, compute, start_store, wait_store,
              src_h, dst_h)
    plsc.subcore_barrier()

    pltpu.sync_copy(s2l, sp_h.at[wid])


@functools.partial(
    pl.kernel,
    out_type=(_sds((E * LW,)), _sds((NW, H * N))),
    mesh=_MESH,
    scratch_types=[
        pltpu.VMEM((BLK,), jnp.int32),
        pltpu.VMEM((BLK,), jnp.int32),
        [pltpu.VMEM((CHA, D), _F32) for _ in range(2)],
        [pltpu.VMEM((CHA, D), _F32) for _ in range(2)],
        [pltpu.VMEM((CHA * LW,), _F32) for _ in range(2)],
        [pltpu.SemaphoreType.DMA for _ in range(2)],
        [pltpu.SemaphoreType.DMA for _ in range(2)],
        pltpu.VMEM((H * N,), _F32),
    ],
    compiler_params=_SC_PARAMS,
)
def _l2_pass1(src_h, dst_h, ast_h, adt_h, z8N_h, exh_h, sp_h, *rest):
    _l2a_body(src_h, dst_h, ast_h, adt_h, z8N_h, exh_h, sp_h, *rest)


CH2B = 16
BLK2 = 256


def _l2b_body(src_h, dst_h, h2_h, exh_h, rt_h, z128_h, accp_h,
              srcb, dstb, wbb, dstv, rg, rows, msg, gsem, ssem, acc_sh):
    cid, sid, wid = _worker()

    @pl.when(sid == 0)
    def _():
        pltpu.sync_copy(z128_h, acc_sh)

    plsc.subcore_barrier()

    def start_gather(c, b):
        idx = srcb.at[pl.ds(pl.multiple_of(c * CH2B, 8), CH2B)]
        didx = dstb.at[pl.ds(pl.multiple_of(c * CH2B, 8), CH2B)]
        pltpu.async_copy(h2_h.at[idx], rows[b], gsem[b])
        pltpu.async_copy(rt_h.at[didx], rg[b], gsem[b])

    def wait_gather(b):
        pltpu.make_async_copy(h2_h.at[srcb.at[pl.ds(0, CH2B)]], rows[b],
                              gsem[b]).wait()
        pltpu.make_async_copy(rt_h.at[dstb.at[pl.ds(0, CH2B)]], rg[b],
                              gsem[b]).wait()

    def start_scatter(c, b, base):
        dstv[b][...] = dstb[pl.ds(pl.multiple_of(c * CH2B, 8), CH2B)]
        pltpu.async_copy(msg[b], acc_sh.at[dstv[b]], ssem[b], add=True)

    def wait_scatter(b):
        pltpu.make_async_copy(msg[b], acc_sh.at[dstv[b]], ssem[b]).wait()

    def compute_chunk(c, b, base):
        def combine(e, _):
            off = pl.multiple_of(c * CH2B * LW + e * LW, LW)
            w16 = wbb[pl.ds(off, LW)] * rg[b][e, pl.ds(0, LW)]
            wv = [_bcast_lane(w16, h) for h in range(H)]
            for cj in range(D // LW):
                acc = wv[0] * rows[b][e, pl.ds(cj * LW, LW)]
                for h in range(1, H):
                    acc = acc + wv[h] * rows[b][e, pl.ds(h * D + cj * LW, LW)]
                msg[b][e, pl.ds(cj * LW, LW)] = acc
            return 0

        lax.fori_loop(0, CH2B, combine, 0)

    def load_extra(base):
        pltpu.sync_copy(exh_h.at[pl.ds(pl.multiple_of(base * LW, 8),
                                       BLK2 * LW)], wbb)

    def sg(c, b):
        start_gather(c, b)

    _pipeline(wid, BLK2, BLK2 // CH2B, srcb, dstb, load_extra, sg,
              wait_gather, compute_chunk, start_scatter, wait_scatter,
              src_h, dst_h)
    plsc.subcore_barrier()

    _dump_shared(cid, sid, acc_sh, accp_h)


@functools.partial(
    pl.kernel,
    out_type=_sds((NC, N, D)),
    mesh=_MESH,
    scratch_types=[
        pltpu.VMEM((BLK2,), jnp.int32),
        pltpu.VMEM((BLK2,), jnp.int32),
        pltpu.VMEM((BLK2 * LW,), _F32),
        [pltpu.VMEM((CH2B,), jnp.int32) for _ in range(2)],
        [pltpu.VMEM((CH2B, D), _F32) for _ in range(2)],
        [pltpu.VMEM((CH2B, H * D), _F32) for _ in range(2)],
        [pltpu.VMEM((CH2B, D), _F32) for _ in range(2)],
        [pltpu.SemaphoreType.DMA for _ in range(2)],
        [pltpu.SemaphoreType.DMA for _ in range(2)],
        pltpu.VMEM_SHARED((N, D), _F32),
    ],
    compiler_params=_SC_PARAMS,
)
def _l2_pass2(src_h, dst_h, h2_h, exh_h, rt_h, z128_h, accp_h, *rest):
    _l2b_body(src_h, dst_h, h2_h, exh_h, rt_h, z128_h, accp_h, *rest)


def kernel(x, edge_index, W1, att_src1, att_dst1, b1, W2, att_src2, att_dst2,
           b2):
    src = edge_index[0].astype(jnp.int32)
    dst = edge_index[1].astype(jnp.int32)
    z128 = jnp.zeros((N, D), _F32)
    zN = jnp.zeros((N,), _F32)
    z8N = jnp.zeros((H * N,), _F32)

    avs = jnp.broadcast_to(att_src1.reshape(D, 1), (D, D))
    avd = jnp.broadcast_to(att_dst1.reshape(D, 1), (D, D))
    h1, ast1, adt1 = _dense1(x, W1, avs, avd)

    accp1, sp1 = _l1_edge(src, dst, h1, ast1, adt1, z128, zN)

    g2 = np.zeros((H, D), np.float32)
    for c in range(D):
        g2[c % H, c] = 1.0
    G2 = jnp.asarray(g2)
    ones32 = jnp.ones((NW, 1), _F32)
    AS2 = att_src2.reshape(H, D).T
    AD2 = att_dst2.reshape(H, D).T
    h2, ast2, adt2 = _dense2(accp1, sp1.T, ones32, b1.reshape(1, D), W2, AS2,
                             AD2, G2)

    sel = np.zeros((NW * H, H), np.float32)
    for k in range(NW * H):
        sel[k, k % H] = 1.0
    SEL = jnp.asarray(sel)
    exh, sp2 = _l2_pass1(src, dst, ast2, adt2, z8N)
    sp2T = sp2.reshape(NW, H, N).transpose(2, 0, 1).reshape(N, NW * H)
    rt2 = _recip(sp2T, SEL, G2)
    accp2 = _l2_pass2(src, dst, h2, exh, rt2, z128)

    return _final(accp2, b2.reshape(1, D))


# speedup vs baseline: 1.0108x; 1.0108x over previous
import functools

import jax
import jax.numpy as jnp
import numpy as np
from jax import lax
from jax.experimental import pallas as pl
from jax.experimental.pallas import tpu as pltpu
from jax.experimental.pallas import tpu_sc as plsc

N = 10000
E = 320000
D = 128
H = 8
LW = 16
NC, NS = 2, 16
NW = NC * NS
EPW = E // NW
RPS = 624
TAIL = N - NS * RPS

TB = 1000
TG = N // TB

_F32 = jnp.float32


def _sds(shape):
    return jax.ShapeDtypeStruct(shape, _F32)


def _dense1_body(x_ref, w_ref, avs_ref, avd_ref, h_ref, as_ref, ad_ref):
    h = jnp.dot(x_ref[...], w_ref[...], preferred_element_type=_F32)
    h_ref[...] = h
    as_ref[...] = jnp.dot(h, avs_ref[...], preferred_element_type=_F32)
    ad_ref[...] = jnp.dot(h, avd_ref[...], preferred_element_type=_F32)


def _dense1(x, W1, avs, avd):
    return pl.pallas_call(
        _dense1_body,
        grid=(TG,),
        in_specs=[
            pl.BlockSpec((TB, D), lambda i: (i, 0)),
            pl.BlockSpec((D, D), lambda i: (0, 0)),
            pl.BlockSpec((D, D), lambda i: (0, 0)),
            pl.BlockSpec((D, D), lambda i: (0, 0)),
        ],
        out_specs=[
            pl.BlockSpec((TB, D), lambda i: (i, 0)),
            pl.BlockSpec((TB, D), lambda i: (i, 0)),
            pl.BlockSpec((TB, D), lambda i: (i, 0)),
        ],
        out_shape=[_sds((N, D)), _sds((N, D)), _sds((N, D))],
    )(x, W1, avs, avd)


def _dense2_body(accp_ref, sp_ref, ones_ref, b1_ref, w2_ref, as2_ref,
                 ad2_ref, g2_ref, h2_ref, as_ref, ad_ref):
    acc = accp_ref[0] + accp_ref[1]
    s = jnp.dot(sp_ref[...], ones_ref[...], preferred_element_type=_F32)
    g = jnp.maximum(acc / (s + 1e-16) + b1_ref[...], 0.0)
    h2 = jnp.dot(g, w2_ref[...], preferred_element_type=_F32)
    h2_ref[...] = h2
    avals = []
    dvals = []
    for h in range(H):
        h2h = h2[:, h * D:(h + 1) * D]
        avals.append(jnp.dot(h2h, as2_ref[:, h:h + 1],
                             preferred_element_type=_F32))
        dvals.append(jnp.dot(h2h, ad2_ref[:, h:h + 1],
                             preferred_element_type=_F32))
    a8 = jnp.concatenate(avals, axis=1)
    d8 = jnp.concatenate(dvals, axis=1)
    as_ref[...] = jnp.dot(a8, g2_ref[...], preferred_element_type=_F32)
    ad_ref[...] = jnp.dot(d8, g2_ref[...], preferred_element_type=_F32)


def _dense2(accp, sp, ones32, b1, W2, AS2, AD2, G2):
    return pl.pallas_call(
        _dense2_body,
        grid=(TG,),
        in_specs=[
            pl.BlockSpec((NC, TB, D), lambda i: (0, i, 0)),
            pl.BlockSpec((TB, NW), lambda i: (i, 0)),
            pl.BlockSpec((NW, 1), lambda i: (0, 0)),
            pl.BlockSpec((1, D), lambda i: (0, 0)),
            pl.BlockSpec((D, H * D), lambda i: (0, 0)),
            pl.BlockSpec((D, H), lambda i: (0, 0)),
            pl.BlockSpec((D, H), lambda i: (0, 0)),
            pl.BlockSpec((H, D), lambda i: (0, 0)),
        ],
        out_specs=[
            pl.BlockSpec((TB, H * D), lambda i: (i, 0)),
            pl.BlockSpec((TB, D), lambda i: (i, 0)),
            pl.BlockSpec((TB, D), lambda i: (i, 0)),
        ],
        out_shape=[_sds((N, H * D)), _sds((N, D)), _sds((N, D))],
    )(accp, sp, ones32, b1, W2, AS2, AD2, G2)


def _recip_body(sp_ref, sel_ref, g2_ref, r_ref):
    s8 = jnp.dot(sp_ref[...], sel_ref[...], preferred_element_type=_F32)
    r8 = 1.0 / (s8 + 1e-16)
    r_ref[...] = jnp.dot(r8, g2_ref[...], preferred_element_type=_F32)


def _recip(spT, SEL, G2):
    return pl.pallas_call(
        _recip_body,
        grid=(TG,),
        in_specs=[
            pl.BlockSpec((TB, NW * H), lambda i: (i, 0)),
            pl.BlockSpec((NW * H, H), lambda i: (0, 0)),
            pl.BlockSpec((H, D), lambda i: (0, 0)),
        ],
        out_specs=pl.BlockSpec((TB, D), lambda i: (i, 0)),
        out_shape=_sds((N, D)),
    )(spT, SEL, G2)


def _final_body(accp_ref, b2_ref, o_ref):
    o_ref[...] = (accp_ref[0] + accp_ref[1]) * (1.0 / H) + b2_ref[...]


def _final(accp, b2):
    return pl.pallas_call(
        _final_body,
        grid=(TG,),
        in_specs=[
            pl.BlockSpec((NC, TB, D), lambda i: (0, i, 0)),
            pl.BlockSpec((1, D), lambda i: (0, 0)),
        ],
        out_specs=pl.BlockSpec((TB, D), lambda i: (i, 0)),
        out_shape=_sds((N, D)),
    )(accp, b2)


_MESH = plsc.VectorSubcoreMesh(core_axis_name="c", subcore_axis_name="s")
_SC_PARAMS = pltpu.CompilerParams(needs_layout_passes=False)

_GDN = lax.GatherDimensionNumbers(offset_dims=(), collapsed_slice_dims=(0,),
                                  start_index_map=(0,))


def _worker():
    cid = lax.axis_index("c")
    sid = lax.axis_index("s")
    return cid, sid, sid * NC + cid


def _bcast_lane(v, l):
    idx = jnp.full((LW, 1), l, jnp.int32)
    return lax.gather(v, idx, _GDN, (1,),
                      mode=lax.GatherScatterMode.PROMISE_IN_BOUNDS)


def _edge_coeffs(asg, adg, exb, ch):
    def body(e, _):
        es = asg[e, pl.ds(0, LW)] + adg[e, pl.ds(0, LW)]
        es = jnp.maximum(es, es * 0.2)
        exb[pl.ds(pl.multiple_of(e * LW, LW), LW)] = jnp.exp(es)
        return 0

    lax.fori_loop(0, ch, body, 0)


def _dump_shared(cid, sid, shared, out_h):
    pltpu.sync_copy(shared.at[pl.ds(sid * RPS, RPS)],
                    out_h.at[cid, pl.ds(sid * RPS, RPS)])

    @pl.when(sid == 0)
    def _():
        pltpu.sync_copy(shared.at[pl.ds(NS * RPS, TAIL)],
                        out_h.at[cid, pl.ds(NS * RPS, TAIL)])


CH = 16
BLK = 512
NBLK = E // BLK


def _block_range(wid, total_blocks):
    q, r = divmod(total_blocks, NW)
    nblk = jnp.where(wid < r, q + 1, q)
    start = jnp.where(wid < r, (q + 1) * wid, q * wid + r)
    return start, nblk


def _pipeline(wid, blk, nchi, srcb, dstb, load_extra, start_gathers,
              wait_gathers, compute, start_store, wait_store, src_h, dst_h):
    start, nblk = _block_range(wid, E // blk)

    def step(c, b, base, prefetch):
        b2 = 1 - b
        if prefetch:
            if isinstance(c, int):
                if c >= 1:
                    wait_store(b2)
            else:
                @pl.when(c >= 1)
                def _():
                    wait_store(b2)
            start_gathers(c + 1, b2)

        wait_gathers(b)
        compute(c, b, base)
        start_store(c, b, base)

    def block(s, _):
        base = pl.multiple_of((start + s) * blk, 8)
        pltpu.sync_copy(src_h.at[pl.ds(base, blk)], srcb)
        pltpu.sync_copy(dst_h.at[pl.ds(base, blk)], dstb)
        load_extra(base)
        start_gathers(0, 0)

        def pair(cc, _):
            for k in range(2):
                step(cc * 2 + k, k, base, True)
            return 0

        lax.fori_loop(0, nchi // 2 - 1, pair, 0)
        step(nchi - 2, 0, base, True)
        step(nchi - 1, 1, base, False)
        wait_store(0)
        wait_store(1)
        return 0

    lax.fori_loop(0, nblk, block, 0)


def _l1_body(src_h, dst_h, h1_h, ast_h, adt_h, z128_h, zN_h, accp_h, sp_h,
             srcb, dstb, dstv, asg, adg, rows, gsem, ssem, s1l, acc_sh):
    cid, sid, wid = _worker()

    pltpu.sync_copy(zN_h, s1l)

    @pl.when(sid == 0)
    def _():
        pltpu.sync_copy(z128_h, acc_sh)

    plsc.subcore_barrier()

    lane0 = lax.iota(jnp.int32, LW) == 0

    def sslice(c):
        return srcb.at[pl.ds(pl.multiple_of(c * CH, 8), CH)]

    def dslice(c):
        return dstb.at[pl.ds(pl.multiple_of(c * CH, 8), CH)]

    def start_gathers(c, b):
        pltpu.async_copy(ast_h.at[sslice(c)], asg[b], gsem[b])
        pltpu.async_copy(adt_h.at[dslice(c)], adg[b], gsem[b])
        pltpu.async_copy(h1_h.at[sslice(c)], rows[b], gsem[b])

    def wait_gathers(b):
        pltpu.make_async_copy(ast_h.at[sslice(0)], asg[b], gsem[b]).wait()
        pltpu.make_async_copy(adt_h.at[dslice(0)], adg[b], gsem[b]).wait()
        pltpu.make_async_copy(h1_h.at[sslice(0)], rows[b], gsem[b]).wait()

    def compute(c, b, base):
        dst16 = dstb[pl.ds(pl.multiple_of(c * CH, 8), CH)]
        dstv[b][...] = dst16
        for l in range(CH):
            es = asg[b][l, pl.ds(0, LW)] + adg[b][l, pl.ds(0, LW)]
            es = jnp.maximum(es, es * 0.2)
            exv = jnp.exp(es)
            for j2 in range(D // LW):
                rows[b][l, pl.ds(j2 * LW, LW)] = (
                    rows[b][l, pl.ds(j2 * LW, LW)] * exv)
            db = _bcast_lane(dst16, l)
            plsc.addupdate_scatter(s1l, [db], exv, mask=lane0)

    def start_store(c, b, base):
        pltpu.async_copy(rows[b], acc_sh.at[dstv[b]], ssem[b], add=True)

    def wait_store(b):
        pltpu.make_async_copy(rows[b], acc_sh.at[dstv[b]], ssem[b]).wait()

    _pipeline(wid, BLK, BLK // CH, srcb, dstb, lambda base: None,
              start_gathers, wait_gathers, compute, start_store, wait_store,
              src_h, dst_h)
    plsc.subcore_barrier()

    _dump_shared(cid, sid, acc_sh, accp_h)
    pltpu.sync_copy(s1l, sp_h.at[wid])


@functools.partial(
    pl.kernel,
    out_type=(_sds((NC, N, D)), _sds((NW, N))),
    mesh=_MESH,
    scratch_types=[
        pltpu.VMEM((BLK,), jnp.int32),
        pltpu.VMEM((BLK,), jnp.int32),
        [pltpu.VMEM((CH,), jnp.int32) for _ in range(2)],
        [pltpu.VMEM((CH, D), _F32) for _ in range(2)],
        [pltpu.VMEM((CH, D), _F32) for _ in range(2)],
        [pltpu.VMEM((CH, D), _F32) for _ in range(2)],
        [pltpu.SemaphoreType.DMA for _ in range(2)],
        [pltpu.SemaphoreType.DMA for _ in range(2)],
        pltpu.VMEM((N,), _F32),
        pltpu.VMEM_SHARED((N, D), _F32),
    ],
    compiler_params=_SC_PARAMS,
)
def _l1_edge(src_h, dst_h, h1_h, ast_h, adt_h, z128_h, zN_h, accp_h, sp_h,
             *rest):
    _l1_body(src_h, dst_h, h1_h, ast_h, adt_h, z128_h, zN_h, accp_h, sp_h,
             *rest)


CHA = 32


def _l2a_body(src_h, dst_h, ast_h, adt_h, z8N_h, exh_h, sp_h,
              srcb, dstb, asg, adg, exb, gsem, esem, s2l):
    cid, sid, wid = _worker()

    pltpu.sync_copy(z8N_h, s2l)
    plsc.subcore_barrier()

    hoff = (lax.iota(jnp.int32, LW) % H) * N
    hmask = lax.iota(jnp.int32, LW) < H

    def sslice(c):
        return srcb.at[pl.ds(pl.multiple_of(c * CHA, 8), CHA)]

    def dslice(c):
        return dstb.at[pl.ds(pl.multiple_of(c * CHA, 8), CHA)]

    def start_gathers(c, b):
        pltpu.async_copy(ast_h.at[sslice(c)], asg[b], gsem[b])
        pltpu.async_copy(adt_h.at[dslice(c)], adg[b], gsem[b])

    def wait_gathers(b):
        pltpu.make_async_copy(ast_h.at[sslice(0)], asg[b], gsem[b]).wait()
        pltpu.make_async_copy(adt_h.at[dslice(0)], adg[b], gsem[b]).wait()

    def compute(c, b, base):
        for g in range(CHA // LW):
            dst16 = dstb[pl.ds(pl.multiple_of(c * CHA + g * LW, 8), LW)]
            for l in range(LW):
                e = g * LW + l
                es = asg[b][e, pl.ds(0, LW)] + adg[b][e, pl.ds(0, LW)]
                es = jnp.maximum(es, es * 0.2)
                exv = jnp.exp(es)
                exb[b][pl.ds(e * LW, LW)] = exv
                db = _bcast_lane(dst16, l)
                plsc.addupdate_scatter(s2l, [hoff + db], exv, mask=hmask)

    def exh_at(c, base):
        off = pl.multiple_of((base + c * CHA) * LW, 8)
        return exh_h.at[pl.ds(off, CHA * LW)]

    def start_store(c, b, base):
        pltpu.async_copy(exb[b], exh_at(c, base), esem[b])

    def wait_store(b):
        pltpu.make_async_copy(exb[b], exh_h.at[pl.ds(0, CHA * LW)],
                              esem[b]).wait()

    _pipeline(wid, BLK, BLK // CHA, srcb, dstb, lambda base: None,
              start_gathers, wait_gathers, compute, start_store, wait_store,
              src_h, dst_h)
    plsc.subcore_barrier()

    pltpu.sync_copy(s2l, sp_h.at[wid])


@functools.partial(
    pl.kernel,
    out_type=(_sds((E * LW,)), _sds((NW, H * N))),
    mesh=_MESH,
    scratch_types=[
        pltpu.VMEM((BLK,), jnp.int32),
        pltpu.VMEM((BLK,), jnp.int32),
        [pltpu.VMEM((CHA, D), _F32) for _ in range(2)],
        [pltpu.VMEM((CHA, D), _F32) for _ in range(2)],
        [pltpu.VMEM((CHA * LW,), _F32) for _ in range(2)],
        [pltpu.SemaphoreType.DMA for _ in range(2)],
        [pltpu.SemaphoreType.DMA for _ in range(2)],
        pltpu.VMEM((H * N,), _F32),
    ],
    compiler_params=_SC_PARAMS,
)
def _l2_pass1(src_h, dst_h, ast_h, adt_h, z8N_h, exh_h, sp_h, *rest):
    _l2a_body(src_h, dst_h, ast_h, adt_h, z8N_h, exh_h, sp_h, *rest)


CH2B = 16
BLK2 = 256


def _l2b_body(src_h, dst_h, h2_h, exh_h, rt_h, z128_h, accp_h,
              srcb, dstb, wbb, dstv, rg, rows, msg, gsem, ssem, acc_sh):
    cid, sid, wid = _worker()

    @pl.when(sid == 0)
    def _():
        pltpu.sync_copy(z128_h, acc_sh)

    plsc.subcore_barrier()

    def start_gather(c, b):
        idx = srcb.at[pl.ds(pl.multiple_of(c * CH2B, 8), CH2B)]
        didx = dstb.at[pl.ds(pl.multiple_of(c * CH2B, 8), CH2B)]
        pltpu.async_copy(h2_h.at[idx], rows[b], gsem[b])
        pltpu.async_copy(rt_h.at[didx], rg[b], gsem[b])

    def wait_gather(b):
        pltpu.make_async_copy(h2_h.at[srcb.at[pl.ds(0, CH2B)]], rows[b],
                              gsem[b]).wait()
        pltpu.make_async_copy(rt_h.at[dstb.at[pl.ds(0, CH2B)]], rg[b],
                              gsem[b]).wait()

    def start_scatter(c, b, base):
        dstv[b][...] = dstb[pl.ds(pl.multiple_of(c * CH2B, 8), CH2B)]
        pltpu.async_copy(msg[b], acc_sh.at[dstv[b]], ssem[b], add=True)

    def wait_scatter(b):
        pltpu.make_async_copy(msg[b], acc_sh.at[dstv[b]], ssem[b]).wait()

    def compute_chunk(c, b, base):
        def combine(e, _):
            off = pl.multiple_of(c * CH2B * LW + e * LW, LW)
            w16 = wbb[pl.ds(off, LW)] * rg[b][e, pl.ds(0, LW)]
            wv = [_bcast_lane(w16, h) for h in range(H)]
            for cj in range(D // LW):
                acc = wv[0] * rows[b][e, pl.ds(cj * LW, LW)]
                for h in range(1, H):
                    acc = acc + wv[h] * rows[b][e, pl.ds(h * D + cj * LW, LW)]
                msg[b][e, pl.ds(cj * LW, LW)] = acc
            return 0

        lax.fori_loop(0, CH2B, combine, 0)

    def load_extra(base):
        pltpu.sync_copy(exh_h.at[pl.ds(pl.multiple_of(base * LW, 8),
                                       BLK2 * LW)], wbb)

    def sg(c, b):
        start_gather(c, b)

    _pipeline(wid, BLK2, BLK2 // CH2B, srcb, dstb, load_extra, sg,
              wait_gather, compute_chunk, start_scatter, wait_scatter,
              src_h, dst_h)
    plsc.subcore_barrier()

    _dump_shared(cid, sid, acc_sh, accp_h)


@functools.partial(
    pl.kernel,
    out_type=_sds((NC, N, D)),
    mesh=_MESH,
    scratch_types=[
        pltpu.VMEM((BLK2,), jnp.int32),
        pltpu.VMEM((BLK2,), jnp.int32),
        pltpu.VMEM((BLK2 * LW,), _F32),
        [pltpu.VMEM((CH2B,), jnp.int32) for _ in range(2)],
        [pltpu.VMEM((CH2B, D), _F32) for _ in range(2)],
        [pltpu.VMEM((CH2B, H * D), _F32) for _ in range(2)],
        [pltpu.VMEM((CH2B, D), _F32) for _ in range(2)],
        [pltpu.SemaphoreType.DMA for _ in range(2)],
        [pltpu.SemaphoreType.DMA for _ in range(2)],
        pltpu.VMEM_SHARED((N, D), _F32),
    ],
    compiler_params=_SC_PARAMS,
)
def _l2_pass2(src_h, dst_h, h2_h, exh_h, rt_h, z128_h, accp_h, *rest):
    _l2b_body(src_h, dst_h, h2_h, exh_h, rt_h, z128_h, accp_h, *rest)


def kernel(x, edge_index, W1, att_src1, att_dst1, b1, W2, att_src2, att_dst2,
           b2):
    src = edge_index[0].astype(jnp.int32)
    dst = edge_index[1].astype(jnp.int32)
    z128 = jnp.zeros((N, D), _F32)
    zN = jnp.zeros((N,), _F32)
    z8N = jnp.zeros((H * N,), _F32)

    avs = jnp.broadcast_to(att_src1.reshape(D, 1), (D, D))
    avd = jnp.broadcast_to(att_dst1.reshape(D, 1), (D, D))
    h1, ast1, adt1 = _dense1(x, W1, avs, avd)

    accp1, sp1 = _l1_edge(src, dst, h1, ast1, adt1, z128, zN)

    g2 = np.zeros((H, D), np.float32)
    for c in range(D):
        g2[c % H, c] = 1.0
    G2 = jnp.asarray(g2)
    ones32 = jnp.ones((NW, 1), _F32)
    AS2 = att_src2.reshape(H, D).T
    AD2 = att_dst2.reshape(H, D).T
    h2, ast2, adt2 = _dense2(accp1, sp1.T, ones32, b1.reshape(1, D), W2, AS2,
                             AD2, G2)

    sel = np.zeros((NW * H, H), np.float32)
    for k in range(NW * H):
        sel[k, k % H] = 1.0
    SEL = jnp.asarray(sel)
    exh, sp2 = _l2_pass1(src, dst, ast2, adt2, z8N)
    sp2T = sp2.reshape(NW, H, N).transpose(2, 0, 1).reshape(N, NW * H)
    rt2 = _recip(sp2T, SEL, G2)
    accp2 = _l2_pass2(src, dst, h2, exh, rt2, z128)

    return _final(accp2, b2.reshape(1, D))


# speedup vs baseline: 1.0159x; 1.0051x over previous
import functools

import jax
import jax.numpy as jnp
import numpy as np
from jax import lax
from jax.experimental import pallas as pl
from jax.experimental.pallas import tpu as pltpu
from jax.experimental.pallas import tpu_sc as plsc

N = 10000
E = 320000
D = 128
H = 8
LW = 16
NC, NS = 2, 16
NW = NC * NS
EPW = E // NW
RPS = 624
TAIL = N - NS * RPS

TB = 1000
TG = N // TB

_F32 = jnp.float32


def _sds(shape):
    return jax.ShapeDtypeStruct(shape, _F32)


def _dense1_body(x_ref, w_ref, avs_ref, avd_ref, h_ref, as_ref, ad_ref):
    h = jnp.dot(x_ref[...], w_ref[...], preferred_element_type=_F32)
    h_ref[...] = h
    as_ref[...] = jnp.dot(h, avs_ref[...], preferred_element_type=_F32)
    ad_ref[...] = jnp.dot(h, avd_ref[...], preferred_element_type=_F32)


def _dense1(x, W1, avs, avd):
    return pl.pallas_call(
        _dense1_body,
        grid=(TG,),
        in_specs=[
            pl.BlockSpec((TB, D), lambda i: (i, 0)),
            pl.BlockSpec((D, D), lambda i: (0, 0)),
            pl.BlockSpec((D, D), lambda i: (0, 0)),
            pl.BlockSpec((D, D), lambda i: (0, 0)),
        ],
        out_specs=[
            pl.BlockSpec((TB, D), lambda i: (i, 0)),
            pl.BlockSpec((TB, D), lambda i: (i, 0)),
            pl.BlockSpec((TB, D), lambda i: (i, 0)),
        ],
        out_shape=[_sds((N, D)), _sds((N, D)), _sds((N, D))],
    )(x, W1, avs, avd)


def _dense2_body(accp_ref, sp_ref, ones_ref, b1_ref, w2_ref, as2_ref,
                 ad2_ref, g2_ref, h2_ref, as_ref, ad_ref):
    acc = accp_ref[0] + accp_ref[1]
    s = jnp.dot(sp_ref[...], ones_ref[...], preferred_element_type=_F32)
    g = jnp.maximum(acc / (s + 1e-16) + b1_ref[...], 0.0)
    h2 = jnp.dot(g, w2_ref[...], preferred_element_type=_F32)
    h2_ref[...] = h2
    avals = []
    dvals = []
    for h in range(H):
        h2h = h2[:, h * D:(h + 1) * D]
        avals.append(jnp.dot(h2h, as2_ref[:, h:h + 1],
                             preferred_element_type=_F32))
        dvals.append(jnp.dot(h2h, ad2_ref[:, h:h + 1],
                             preferred_element_type=_F32))
    a8 = jnp.concatenate(avals, axis=1)
    d8 = jnp.concatenate(dvals, axis=1)
    as_ref[...] = jnp.dot(a8, g2_ref[...], preferred_element_type=_F32)
    ad_ref[...] = jnp.dot(d8, g2_ref[...], preferred_element_type=_F32)


def _dense2(accp, sp, ones32, b1, W2, AS2, AD2, G2):
    return pl.pallas_call(
        _dense2_body,
        grid=(TG,),
        in_specs=[
            pl.BlockSpec((NC, TB, D), lambda i: (0, i, 0)),
            pl.BlockSpec((TB, NW), lambda i: (i, 0)),
            pl.BlockSpec((NW, 1), lambda i: (0, 0)),
            pl.BlockSpec((1, D), lambda i: (0, 0)),
            pl.BlockSpec((D, H * D), lambda i: (0, 0)),
            pl.BlockSpec((D, H), lambda i: (0, 0)),
            pl.BlockSpec((D, H), lambda i: (0, 0)),
            pl.BlockSpec((H, D), lambda i: (0, 0)),
        ],
        out_specs=[
            pl.BlockSpec((TB, H * D), lambda i: (i, 0)),
            pl.BlockSpec((TB, D), lambda i: (i, 0)),
            pl.BlockSpec((TB, D), lambda i: (i, 0)),
        ],
        out_shape=[_sds((N, H * D)), _sds((N, D)), _sds((N, D))],
    )(accp, sp, ones32, b1, W2, AS2, AD2, G2)


def _recip_body(sp_ref, sel_ref, g2_ref, r_ref):
    s8 = jnp.dot(sp_ref[...], sel_ref[...], preferred_element_type=_F32)
    r8 = 1.0 / (s8 + 1e-16)
    r_ref[...] = jnp.dot(r8, g2_ref[...], preferred_element_type=_F32)


def _recip(spT, SEL, G2):
    return pl.pallas_call(
        _recip_body,
        grid=(TG,),
        in_specs=[
            pl.BlockSpec((TB, NW * H), lambda i: (i, 0)),
            pl.BlockSpec((NW * H, H), lambda i: (0, 0)),
            pl.BlockSpec((H, D), lambda i: (0, 0)),
        ],
        out_specs=pl.BlockSpec((TB, D), lambda i: (i, 0)),
        out_shape=_sds((N, D)),
    )(spT, SEL, G2)


def _final_body(accp_ref, b2_ref, o_ref):
    o_ref[...] = (accp_ref[0] + accp_ref[1]) * (1.0 / H) + b2_ref[...]


def _final(accp, b2):
    return pl.pallas_call(
        _final_body,
        grid=(TG,),
        in_specs=[
            pl.BlockSpec((NC, TB, D), lambda i: (0, i, 0)),
            pl.BlockSpec((1, D), lambda i: (0, 0)),
        ],
        out_specs=pl.BlockSpec((TB, D), lambda i: (i, 0)),
        out_shape=_sds((N, D)),
    )(accp, b2)


_MESH = plsc.VectorSubcoreMesh(core_axis_name="c", subcore_axis_name="s")
_SC_PARAMS = pltpu.CompilerParams(needs_layout_passes=False)

_GDN = lax.GatherDimensionNumbers(offset_dims=(), collapsed_slice_dims=(0,),
                                  start_index_map=(0,))


def _worker():
    cid = lax.axis_index("c")
    sid = lax.axis_index("s")
    return cid, sid, sid * NC + cid


def _bcast_lane(v, l):
    idx = jnp.full((LW, 1), l, jnp.int32)
    return lax.gather(v, idx, _GDN, (1,),
                      mode=lax.GatherScatterMode.PROMISE_IN_BOUNDS)


def _edge_coeffs(asg, adg, exb, ch):
    def body(e, _):
        es = asg[e, pl.ds(0, LW)] + adg[e, pl.ds(0, LW)]
        es = jnp.maximum(es, es * 0.2)
        exb[pl.ds(pl.multiple_of(e * LW, LW), LW)] = jnp.exp(es)
        return 0

    lax.fori_loop(0, ch, body, 0)


def _dump_shared(cid, sid, shared, out_h):
    pltpu.sync_copy(shared.at[pl.ds(sid * RPS, RPS)],
                    out_h.at[cid, pl.ds(sid * RPS, RPS)])

    @pl.when(sid == 0)
    def _():
        pltpu.sync_copy(shared.at[pl.ds(NS * RPS, TAIL)],
                        out_h.at[cid, pl.ds(NS * RPS, TAIL)])


CH = 16
BLK = 512
NBLK = E // BLK


def _block_range(wid, total_blocks):
    q, r = divmod(total_blocks, NW)
    nblk = jnp.where(wid < r, q + 1, q)
    start = jnp.where(wid < r, (q + 1) * wid, q * wid + r)
    return start, nblk


def _pipeline4(wid, blk, nchi, srcb, dstb, load_extra, start_gathers,
               wait_gathers, compute, start_store, wait_store, src_h, dst_h):
    start, nblk = _block_range(wid, E // blk)
    NB = 4

    def block(s, _):
        base = pl.multiple_of((start + s) * blk, 8)
        pltpu.sync_copy(src_h.at[pl.ds(base, blk)], srcb)
        pltpu.sync_copy(dst_h.at[pl.ds(base, blk)], dstb)
        load_extra(base)
        for b in range(NB - 1):
            start_gathers(b, b)

        def quad(cc, _):
            for k in range(NB):
                c = cc * NB + k
                bw = (k + NB - 1) % NB

                def waiter():
                    wait_store(bw)

                if k == 0:
                    pl.when(cc >= 1)(waiter)
                else:
                    waiter()

                @pl.when(c + NB - 1 < nchi)
                def _():
                    start_gathers(c + NB - 1, bw)

                wait_gathers(k)
                compute(c, k, base)
                start_store(c, k, base)
            return 0

        lax.fori_loop(0, nchi // NB, quad, 0)
        wait_store((nchi - 1) % NB)
        return 0

    lax.fori_loop(0, nblk, block, 0)


def _pipeline(wid, blk, nchi, srcb, dstb, load_extra, start_gathers,
              wait_gathers, compute, start_store, wait_store, src_h, dst_h):
    start, nblk = _block_range(wid, E // blk)

    def step(c, b, base, prefetch):
        b2 = 1 - b
        if prefetch:
            if isinstance(c, int):
                if c >= 1:
                    wait_store(b2)
            else:
                @pl.when(c >= 1)
                def _():
                    wait_store(b2)
            start_gathers(c + 1, b2)

        wait_gathers(b)
        compute(c, b, base)
        start_store(c, b, base)

    def block(s, _):
        base = pl.multiple_of((start + s) * blk, 8)
        pltpu.sync_copy(src_h.at[pl.ds(base, blk)], srcb)
        pltpu.sync_copy(dst_h.at[pl.ds(base, blk)], dstb)
        load_extra(base)
        start_gathers(0, 0)

        def pair(cc, _):
            for k in range(2):
                step(cc * 2 + k, k, base, True)
            return 0

        lax.fori_loop(0, nchi // 2 - 1, pair, 0)
        step(nchi - 2, 0, base, True)
        step(nchi - 1, 1, base, False)
        wait_store(0)
        wait_store(1)
        return 0

    lax.fori_loop(0, nblk, block, 0)


def _l1_body(src_h, dst_h, h1_h, ast_h, adt_h, z128_h, zN_h, accp_h, sp_h,
             srcb, dstb, dstv, asg, adg, rows, gsem, ssem, s1l, acc_sh):
    cid, sid, wid = _worker()

    pltpu.sync_copy(zN_h, s1l)

    @pl.when(sid == 0)
    def _():
        pltpu.sync_copy(z128_h, acc_sh)

    plsc.subcore_barrier()

    lane0 = lax.iota(jnp.int32, LW) == 0

    def sslice(c):
        return srcb.at[pl.ds(pl.multiple_of(c * CH, 8), CH)]

    def dslice(c):
        return dstb.at[pl.ds(pl.multiple_of(c * CH, 8), CH)]

    def start_gathers(c, b):
        pltpu.async_copy(ast_h.at[sslice(c)], asg[b], gsem[b])
        pltpu.async_copy(adt_h.at[dslice(c)], adg[b], gsem[b])
        pltpu.async_copy(h1_h.at[sslice(c)], rows[b], gsem[b])

    def wait_gathers(b):
        pltpu.make_async_copy(ast_h.at[sslice(0)], asg[b], gsem[b]).wait()
        pltpu.make_async_copy(adt_h.at[dslice(0)], adg[b], gsem[b]).wait()
        pltpu.make_async_copy(h1_h.at[sslice(0)], rows[b], gsem[b]).wait()

    def compute(c, b, base):
        dst16 = dstb[pl.ds(pl.multiple_of(c * CH, 8), CH)]
        dstv[b][...] = dst16
        for l in range(CH):
            es = asg[b][l, pl.ds(0, LW)] + adg[b][l, pl.ds(0, LW)]
            es = jnp.maximum(es, es * 0.2)
            exv = jnp.exp(es)
            for j2 in range(D // LW):
                rows[b][l, pl.ds(j2 * LW, LW)] = (
                    rows[b][l, pl.ds(j2 * LW, LW)] * exv)
            db = _bcast_lane(dst16, l)
            plsc.addupdate_scatter(s1l, [db], exv, mask=lane0)

    def start_store(c, b, base):
        pltpu.async_copy(rows[b], acc_sh.at[dstv[b]], ssem[b], add=True)

    def wait_store(b):
        pltpu.make_async_copy(rows[b], acc_sh.at[dstv[b]], ssem[b]).wait()

    _pipeline4(wid, BLK, BLK // CH, srcb, dstb, lambda base: None,
               start_gathers, wait_gathers, compute, start_store, wait_store,
               src_h, dst_h)
    plsc.subcore_barrier()

    _dump_shared(cid, sid, acc_sh, accp_h)
    pltpu.sync_copy(s1l, sp_h.at[wid])


@functools.partial(
    pl.kernel,
    out_type=(_sds((NC, N, D)), _sds((NW, N))),
    mesh=_MESH,
    scratch_types=[
        pltpu.VMEM((BLK,), jnp.int32),
        pltpu.VMEM((BLK,), jnp.int32),
        [pltpu.VMEM((CH,), jnp.int32) for _ in range(4)],
        [pltpu.VMEM((CH, D), _F32) for _ in range(4)],
        [pltpu.VMEM((CH, D), _F32) for _ in range(4)],
        [pltpu.VMEM((CH, D), _F32) for _ in range(4)],
        [pltpu.SemaphoreType.DMA for _ in range(4)],
        [pltpu.SemaphoreType.DMA for _ in range(4)],
        pltpu.VMEM((N,), _F32),
        pltpu.VMEM_SHARED((N, D), _F32),
    ],
    compiler_params=_SC_PARAMS,
)
def _l1_edge(src_h, dst_h, h1_h, ast_h, adt_h, z128_h, zN_h, accp_h, sp_h,
             *rest):
    _l1_body(src_h, dst_h, h1_h, ast_h, adt_h, z128_h, zN_h, accp_h, sp_h,
             *rest)


CHA = 32


def _l2a_body(src_h, dst_h, ast_h, adt_h, z8N_h, exh_h, sp_h,
              srcb, dstb, asg, adg, exb, gsem, esem, s2l):
    cid, sid, wid = _worker()

    pltpu.sync_copy(z8N_h, s2l)
    plsc.subcore_barrier()

    hoff = (lax.iota(jnp.int32, LW) % H) * N
    hmask = lax.iota(jnp.int32, LW) < H

    def sslice(c):
        return srcb.at[pl.ds(pl.multiple_of(c * CHA, 8), CHA)]

    def dslice(c):
        return dstb.at[pl.ds(pl.multiple_of(c * CHA, 8), CHA)]

    def start_gathers(c, b):
        pltpu.async_copy(ast_h.at[sslice(c)], asg[b], gsem[b])
        pltpu.async_copy(adt_h.at[dslice(c)], adg[b], gsem[b])

    def wait_gathers(b):
        pltpu.make_async_copy(ast_h.at[sslice(0)], asg[b], gsem[b]).wait()
        pltpu.make_async_copy(adt_h.at[dslice(0)], adg[b], gsem[b]).wait()

    def compute(c, b, base):
        for g in range(CHA // LW):
            dst16 = dstb[pl.ds(pl.multiple_of(c * CHA + g * LW, 8), LW)]
            for l in range(LW):
                e = g * LW + l
                es = asg[b][e, pl.ds(0, LW)] + adg[b][e, pl.ds(0, LW)]
                es = jnp.maximum(es, es * 0.2)
                exv = jnp.exp(es)
                exb[b][pl.ds(e * LW, LW)] = exv
                db = _bcast_lane(dst16, l)
                plsc.addupdate_scatter(s2l, [hoff + db], exv, mask=hmask)

    def exh_at(c, base):
        off = pl.multiple_of((base + c * CHA) * LW, 8)
        return exh_h.at[pl.ds(off, CHA * LW)]

    def start_store(c, b, base):
        pltpu.async_copy(exb[b], exh_at(c, base), esem[b])

    def wait_store(b):
        pltpu.make_async_copy(exb[b], exh_h.at[pl.ds(0, CHA * LW)],
                              esem[b]).wait()

    _pipeline4(wid, BLK, BLK // CHA, srcb, dstb, lambda base: None,
               start_gathers, wait_gathers, compute, start_store, wait_store,
               src_h, dst_h)
    plsc.subcore_barrier()

    pltpu.sync_copy(s2l, sp_h.at[wid])


@functools.partial(
    pl.kernel,
    out_type=(_sds((E * LW,)), _sds((NW, H * N))),
    mesh=_MESH,
    scratch_types=[
        pltpu.VMEM((BLK,), jnp.int32),
        pltpu.VMEM((BLK,), jnp.int32),
        [pltpu.VMEM((CHA, D), _F32) for _ in range(4)],
        [pltpu.VMEM((CHA, D), _F32) for _ in range(4)],
        [pltpu.VMEM((CHA * LW,), _F32) for _ in range(4)],
        [pltpu.SemaphoreType.DMA for _ in range(4)],
        [pltpu.SemaphoreType.DMA for _ in range(4)],
        pltpu.VMEM((H * N,), _F32),
    ],
    compiler_params=_SC_PARAMS,
)
def _l2_pass1(src_h, dst_h, ast_h, adt_h, z8N_h, exh_h, sp_h, *rest):
    _l2a_body(src_h, dst_h, ast_h, adt_h, z8N_h, exh_h, sp_h, *rest)


CH2B = 16
BLK2 = 256


def _l2b_body(src_h, dst_h, h2_h, exh_h, rt_h, z128_h, accp_h,
              srcb, dstb, wbb, dstv, rg, rows, msg, gsem, ssem, acc_sh):
    cid, sid, wid = _worker()

    @pl.when(sid == 0)
    def _():
        pltpu.sync_copy(z128_h, acc_sh)

    plsc.subcore_barrier()

    def start_gather(c, b):
        idx = srcb.at[pl.ds(pl.multiple_of(c * CH2B, 8), CH2B)]
        didx = dstb.at[pl.ds(pl.multiple_of(c * CH2B, 8), CH2B)]
        pltpu.async_copy(h2_h.at[idx], rows[b], gsem[b])
        pltpu.async_copy(rt_h.at[didx], rg[b], gsem[b])

    def wait_gather(b):
        pltpu.make_async_copy(h2_h.at[srcb.at[pl.ds(0, CH2B)]], rows[b],
                              gsem[b]).wait()
        pltpu.make_async_copy(rt_h.at[dstb.at[pl.ds(0, CH2B)]], rg[b],
                              gsem[b]).wait()

    def start_scatter(c, b, base):
        dstv[b][...] = dstb[pl.ds(pl.multiple_of(c * CH2B, 8), CH2B)]
        pltpu.async_copy(msg[b], acc_sh.at[dstv[b]], ssem[b], add=True)

    def wait_scatter(b):
        pltpu.make_async_copy(msg[b], acc_sh.at[dstv[b]], ssem[b]).wait()

    def compute_chunk(c, b, base):
        def combine(e, _):
            off = pl.multiple_of(c * CH2B * LW + e * LW, LW)
            w16 = wbb[pl.ds(off, LW)] * rg[b][e, pl.ds(0, LW)]
            wv = [_bcast_lane(w16, h) for h in range(H)]
            for cj in range(D // LW):
                acc = wv[0] * rows[b][e, pl.ds(cj * LW, LW)]
                for h in range(1, H):
                    acc = acc + wv[h] * rows[b][e, pl.ds(h * D + cj * LW, LW)]
                msg[b][e, pl.ds(cj * LW, LW)] = acc
            return 0

        lax.fori_loop(0, CH2B, combine, 0)

    def load_extra(base):
        pltpu.sync_copy(exh_h.at[pl.ds(pl.multiple_of(base * LW, 8),
                                       BLK2 * LW)], wbb)

    def sg(c, b):
        start_gather(c, b)

    _pipeline(wid, BLK2, BLK2 // CH2B, srcb, dstb, load_extra, sg,
              wait_gather, compute_chunk, start_scatter, wait_scatter,
              src_h, dst_h)
    plsc.subcore_barrier()

    _dump_shared(cid, sid, acc_sh, accp_h)


@functools.partial(
    pl.kernel,
    out_type=_sds((NC, N, D)),
    mesh=_MESH,
    scratch_types=[
        pltpu.VMEM((BLK2,), jnp.int32),
        pltpu.VMEM((BLK2,), jnp.int32),
        pltpu.VMEM((BLK2 * LW,), _F32),
        [pltpu.VMEM((CH2B,), jnp.int32) for _ in range(2)],
        [pltpu.VMEM((CH2B, D), _F32) for _ in range(2)],
        [pltpu.VMEM((CH2B, H * D), _F32) for _ in range(2)],
        [pltpu.VMEM((CH2B, D), _F32) for _ in range(2)],
        [pltpu.SemaphoreType.DMA for _ in range(2)],
        [pltpu.SemaphoreType.DMA for _ in range(2)],
        pltpu.VMEM_SHARED((N, D), _F32),
    ],
    compiler_params=_SC_PARAMS,
)
def _l2_pass2(src_h, dst_h, h2_h, exh_h, rt_h, z128_h, accp_h, *rest):
    _l2b_body(src_h, dst_h, h2_h, exh_h, rt_h, z128_h, accp_h, *rest)


def kernel(x, edge_index, W1, att_src1, att_dst1, b1, W2, att_src2, att_dst2,
           b2):
    src = edge_index[0].astype(jnp.int32)
    dst = edge_index[1].astype(jnp.int32)
    z128 = jnp.zeros((N, D), _F32)
    zN = jnp.zeros((N,), _F32)
    z8N = jnp.zeros((H * N,), _F32)

    avs = jnp.broadcast_to(att_src1.reshape(D, 1), (D, D))
    avd = jnp.broadcast_to(att_dst1.reshape(D, 1), (D, D))
    h1, ast1, adt1 = _dense1(x, W1, avs, avd)

    accp1, sp1 = _l1_edge(src, dst, h1, ast1, adt1, z128, zN)

    g2 = np.zeros((H, D), np.float32)
    for c in range(D):
        g2[c % H, c] = 1.0
    G2 = jnp.asarray(g2)
    ones32 = jnp.ones((NW, 1), _F32)
    AS2 = att_src2.reshape(H, D).T
    AD2 = att_dst2.reshape(H, D).T
    h2, ast2, adt2 = _dense2(accp1, sp1.T, ones32, b1.reshape(1, D), W2, AS2,
                             AD2, G2)

    sel = np.zeros((NW * H, H), np.float32)
    for k in range(NW * H):
        sel[k, k % H] = 1.0
    SEL = jnp.asarray(sel)
    exh, sp2 = _l2_pass1(src, dst, ast2, adt2, z8N)
    sp2T = sp2.reshape(NW, H, N).transpose(2, 0, 1).reshape(N, NW * H)
    rt2 = _recip(sp2T, SEL, G2)
    accp2 = _l2_pass2(src, dst, h2, exh, rt2, z128)

    return _final(accp2, b2.reshape(1, D))
